# Initial kernel scaffold; baseline (speedup 1.0000x reference)
import jax
import jax.numpy as jnp
from jax.experimental import pallas as pl


def kernel(node_features, edge_index, edge_attr, membership, Wi, bi, We, be, Wel, bel, W1, b1, W2, b2, Wh1, bh1, Wh2, bh2, Wh3, bh3):
    raise NotImplementedError("write your pallas kernel here")


# speedup vs baseline: 3.4911x; 3.4911x over previous
import functools

_INTERPRET = False

import jax
import jax.numpy as jnp
from jax import lax
from jax.experimental import pallas as pl
from jax.experimental.pallas import tpu as pltpu
from jax.experimental.pallas import tpu_sc as plsc

N = 10000
E = 320000
D_IN = 128
H = 128
NF = 512
NPAD = 10240
NC, NS = 2, 16
NTILES = NC * NS
EPT = E // NTILES
K = 80
CHUNKS = EPT // K
RPT = NPAD // NS
BM = 2000
GRID = N // BM


_sc_mesh = plsc.VectorSubcoreMesh(
    core_axis_name="c", subcore_axis_name="s", num_cores=NC, num_subcores=NS)


def _sc_kernel_call(*args):
    return pl.kernel(
        _sc_aggr,
        out_type=jax.ShapeDtypeStruct((NC, NPAD, H), jnp.float32),
        mesh=_sc_mesh,
        interpret=_INTERPRET,
        scratch_types=[
            pltpu.VMEM((K,), jnp.int32),
            pltpu.VMEM((K,), jnp.int32),
            pltpu.VMEM((K,), jnp.float32),
            pltpu.VMEM((K, H), jnp.float32),
            pltpu.VMEM((H,), jnp.float32),
            pltpu.VMEM((H,), jnp.float32),
            pltpu.VMEM_SHARED((NPAD, H), jnp.float32),
            pltpu.SemaphoreType.DMA,
        ],
    )(*args)


def _sc_aggr(x_hbm, src3, dst3, a3, vl_hbm, cl_hbm, out_hbm,
             si_vm, di_vm, a_vm, rows_v, vl_vm, cl_vm, aggr_sh, sem):
    c = lax.axis_index("c")
    s = lax.axis_index("s")
    w = s * NC + c

    pltpu.sync_copy(vl_hbm, vl_vm)
    pltpu.sync_copy(cl_hbm, cl_vm)
    vlv = [vl_vm[pl.ds(16 * g, 16)] for g in range(8)]
    clv = [cl_vm[pl.ds(16 * g, 16)] for g in range(8)]

    z = jnp.zeros((16,), jnp.float32)

    def _zrow(e, carry):
        for g in range(8):
            rows_v[e, pl.ds(16 * g, 16)] = z
        return carry

    lax.fori_loop(0, K, _zrow, 0)
    rbase = s * RPT
    for j in range(RPT // K):
        pltpu.sync_copy(rows_v, aggr_sh.at[pl.ds(rbase + j * K, K)])
    plsc.subcore_barrier()

    def _chunk(j, carry):
        pltpu.sync_copy(src3.at[w, j], si_vm)
        pltpu.sync_copy(dst3.at[w, j], di_vm)
        pltpu.sync_copy(a3.at[w, j], a_vm)
        pltpu.async_copy(x_hbm.at[si_vm], rows_v, sem).wait()

        def _egroup(eg, ecarry):
            av = a_vm[pl.ds(16 * eg, 16)]
            for ei in range(16):
                ab = lax.broadcast(av[ei], (16,))
                e = 16 * eg + ei
                for g in range(8):
                    r = rows_v[e, pl.ds(16 * g, 16)]
                    rows_v[e, pl.ds(16 * g, 16)] = jnp.maximum(
                        r + ab * vlv[g] + clv[g], 0.0)
            return ecarry

        lax.fori_loop(0, K // 16, _egroup, 0)
        pltpu.sync_copy(rows_v, aggr_sh.at[di_vm], add=True)
        return carry

    lax.fori_loop(0, CHUNKS, _chunk, 0)
    plsc.subcore_barrier()

    for j in range(RPT // K):
        sl = pl.ds(rbase + j * K, K)
        pltpu.sync_copy(aggr_sh.at[sl], rows_v)
        pltpu.sync_copy(rows_v, out_hbm.at[c, sl])


def _proj_body(nf_ref, Wi_ref, bi_ref, We_ref, be_ref, Wel_ref, bel_ref,
               x_ref, vc_ref):
    i = pl.program_id(0)
    x_ref[...] = (jnp.dot(nf_ref[...], Wi_ref[...],
                          preferred_element_type=jnp.float32,
                           precision=jax.lax.Precision.HIGHEST) + bi_ref[...])

    @pl.when(i == 0)
    def _():
        for l in range(3):
            Wl = Wel_ref[l]
            vr = jnp.dot(We_ref[...], Wl, preferred_element_type=jnp.float32,
                           precision=jax.lax.Precision.HIGHEST)
            cr = (jnp.dot(be_ref[...], Wl, preferred_element_type=jnp.float32,
                           precision=jax.lax.Precision.HIGHEST)
                  + bel_ref[pl.ds(l, 1), :])
            vc_ref[pl.ds(2 * l, 1), :] = vr
            vc_ref[pl.ds(2 * l + 1, 1), :] = cr
        vc_ref[pl.ds(6, 2), :] = jnp.zeros((2, H), jnp.float32)


def _proj(nf, Wi, bi2, We, be2, Wel, bel8):
    return pl.pallas_call(
        _proj_body,
        interpret=_INTERPRET,
        grid=(GRID,),
        in_specs=[
            pl.BlockSpec((BM, D_IN), lambda i: (i, 0)),
            pl.BlockSpec((D_IN, H), lambda i: (0, 0)),
            pl.BlockSpec((1, H), lambda i: (0, 0)),
            pl.BlockSpec((1, H), lambda i: (0, 0)),
            pl.BlockSpec((1, H), lambda i: (0, 0)),
            pl.BlockSpec((3, H, H), lambda i: (0, 0, 0)),
            pl.BlockSpec((8, H), lambda i: (0, 0)),
        ],
        out_specs=[
            pl.BlockSpec((BM, H), lambda i: (i, 0)),
            pl.BlockSpec((8, H), lambda i: (0, 0)),
        ],
        out_shape=[
            jax.ShapeDtypeStruct((NPAD, H), jnp.float32),
            jax.ShapeDtypeStruct((8, H), jnp.float32),
        ],
    )(nf, Wi, bi2, We, be2, Wel, bel8)


def _mlp_body(x_ref, agg_ref, W1_ref, b1_ref, W2_ref, b2_ref, o_ref):
    h = x_ref[...] + agg_ref[0] + agg_ref[1]
    t = jnp.maximum(jnp.dot(h, W1_ref[...],
                            preferred_element_type=jnp.float32,
                           precision=jax.lax.Precision.HIGHEST) + b1_ref[...],
                    0.0)
    y = jnp.dot(t, W2_ref[...], preferred_element_type=jnp.float32,
                           precision=jax.lax.Precision.HIGHEST) + b2_ref[...]
    o_ref[...] = jnp.maximum(y, 0.0)


def _mlp(x, agg, W1l, b1l2, W2l, b2l2):
    return pl.pallas_call(
        _mlp_body,
        interpret=_INTERPRET,
        grid=(GRID,),
        in_specs=[
            pl.BlockSpec((BM, H), lambda i: (i, 0)),
            pl.BlockSpec((2, BM, H), lambda i: (0, i, 0)),
            pl.BlockSpec((H, H), lambda i: (0, 0)),
            pl.BlockSpec((1, H), lambda i: (0, 0)),
            pl.BlockSpec((H, H), lambda i: (0, 0)),
            pl.BlockSpec((1, H), lambda i: (0, 0)),
        ],
        out_specs=pl.BlockSpec((BM, H), lambda i: (i, 0)),
        out_shape=jax.ShapeDtypeStruct((NPAD, H), jnp.float32),
    )(x, agg, W1l, b1l2, W2l, b2l2)


def _mlp3_body(x_ref, agg_ref, W1_ref, b1_ref, W2_ref, b2_ref, m_ref,
               o_ref, ps_ref, cnt_ref):
    i = pl.program_id(0)
    h = x_ref[...] + agg_ref[0] + agg_ref[1]
    t = jnp.maximum(jnp.dot(h, W1_ref[...],
                            preferred_element_type=jnp.float32,
                           precision=jax.lax.Precision.HIGHEST) + b1_ref[...],
                    0.0)
    y = jnp.dot(t, W2_ref[...], preferred_element_type=jnp.float32,
                           precision=jax.lax.Precision.HIGHEST) + b2_ref[...]
    xo = jnp.maximum(y, 0.0)
    o_ref[...] = xo
    oh = (jax.lax.broadcasted_iota(jnp.int32, (NF, BM), 0)
          == m_ref[0, 0, :][None, :]).astype(jnp.float32)
    ps = jax.lax.dot_general(oh, xo, (((1,), (0,)), ((), ())),
                             preferred_element_type=jnp.float32,
                             precision=jax.lax.Precision.HIGHEST)
    cb = jnp.broadcast_to(jnp.sum(oh, axis=1, keepdims=True), (NF, H))

    @pl.when(i == 0)
    def _():
        ps_ref[...] = ps
        cnt_ref[...] = cb

    @pl.when(i != 0)
    def _():
        ps_ref[...] = ps_ref[...] + ps
        cnt_ref[...] = cnt_ref[...] + cb


def _mlp3(x, agg, W1l, b1l2, W2l, b2l2, m3):
    return pl.pallas_call(
        _mlp3_body,
        interpret=_INTERPRET,
        grid=(GRID,),
        in_specs=[
            pl.BlockSpec((BM, H), lambda i: (i, 0)),
            pl.BlockSpec((2, BM, H), lambda i: (0, i, 0)),
            pl.BlockSpec((H, H), lambda i: (0, 0)),
            pl.BlockSpec((1, H), lambda i: (0, 0)),
            pl.BlockSpec((H, H), lambda i: (0, 0)),
            pl.BlockSpec((1, H), lambda i: (0, 0)),
            pl.BlockSpec((1, 1, BM), lambda i: (i, 0, 0)),
        ],
        out_specs=[
            pl.BlockSpec((BM, H), lambda i: (i, 0)),
            pl.BlockSpec((NF, H), lambda i: (0, 0)),
            pl.BlockSpec((NF, H), lambda i: (0, 0)),
        ],
        out_shape=[
            jax.ShapeDtypeStruct((NPAD, H), jnp.float32),
            jax.ShapeDtypeStruct((NF, H), jnp.float32),
            jax.ShapeDtypeStruct((NF, H), jnp.float32),
        ],
    )(x, agg, W1l, b1l2, W2l, b2l2, m3)


def _head_body(ps_ref, cnt_ref, Wh1_ref, bh1_ref, Wh2_ref, bh2_ref,
               Wh3_ref, bh3_ref, o_ref):
    pooled = ps_ref[...] / jnp.maximum(cnt_ref[...], 1.0)
    ctx = jnp.mean(pooled, axis=0, keepdims=True)
    A = Wh1_ref[pl.ds(0, H), :]
    B = Wh1_ref[pl.ds(H, H), :]
    h1 = jnp.maximum(
        jnp.dot(pooled, A, preferred_element_type=jnp.float32,
                           precision=jax.lax.Precision.HIGHEST)
        + jnp.dot(ctx, B, preferred_element_type=jnp.float32,
                           precision=jax.lax.Precision.HIGHEST)
        + bh1_ref[...], 0.0)
    h2 = jnp.maximum(
        jnp.dot(h1, Wh2_ref[...], preferred_element_type=jnp.float32,
                           precision=jax.lax.Precision.HIGHEST)
        + bh2_ref[...], 0.0)
    o_ref[...] = (jnp.dot(h2, Wh3_ref[...], preferred_element_type=jnp.float32,
                           precision=jax.lax.Precision.HIGHEST)
                  + bh3_ref[...])


def _head(ps, cnt, Wh1, bh12, Wh2, bh22, Wh3, bh32):
    return pl.pallas_call(
        _head_body,
        interpret=_INTERPRET,
        out_shape=jax.ShapeDtypeStruct((NF, 1), jnp.float32),
    )(ps, cnt, Wh1, bh12, Wh2, bh22, Wh3, bh32)


def kernel(node_features, edge_index, edge_attr, membership,
           Wi, bi, We, be, Wel, bel, W1, b1, W2, b2,
           Wh1, bh1, Wh2, bh2, Wh3, bh3):
    f32 = jnp.float32
    nf = node_features.astype(f32)
    bel8 = jnp.concatenate([bel.astype(f32), jnp.zeros((5, H), f32)], axis=0)
    x, vc = _proj(nf, Wi.astype(f32), bi.reshape(1, H).astype(f32),
                  We.astype(f32), be.reshape(1, H).astype(f32),
                  Wel.astype(f32), bel8)

    src3 = edge_index[0].reshape(NTILES, CHUNKS, K)
    dst3 = edge_index[1].reshape(NTILES, CHUNKS, K)
    a3 = edge_attr.astype(f32)[:, 0].reshape(NTILES, CHUNKS, K)
    m3 = membership.reshape(GRID, 1, BM)

    for l in range(3):
        agg = _sc_kernel_call(x, src3, dst3, a3, vc[2 * l], vc[2 * l + 1])
        W1l = W1[l].astype(f32)
        b1l2 = b1[l].reshape(1, H).astype(f32)
        W2l = W2[l].astype(f32)
        b2l2 = b2[l].reshape(1, H).astype(f32)
        if l < 2:
            x = _mlp(x, agg, W1l, b1l2, W2l, b2l2)
        else:
            x, ps, cnt = _mlp3(x, agg, W1l, b1l2, W2l, b2l2, m3)

    return _head(ps, cnt, Wh1.astype(f32), bh1.reshape(1, H).astype(f32),
                 Wh2.astype(f32), bh2.reshape(1, H).astype(f32),
                 Wh3.astype(f32), bh3.reshape(1, 1).astype(f32))


# speedup vs baseline: 8.0025x; 2.2922x over previous
import functools

_INTERPRET = False

import jax
import jax.numpy as jnp
from jax import lax
from jax.experimental import pallas as pl
from jax.experimental.pallas import tpu as pltpu
from jax.experimental.pallas import tpu_sc as plsc

N = 10000
E = 320000
D_IN = 128
H = 128
NF = 512
NPAD = 10240
NC, NS = 2, 16
NTILES = NC * NS
EPT = E // NTILES
K = 80
CHUNKS = EPT // K
RPT = NPAD // NS
BM = 2000
GRID = N // BM


_sc_mesh = plsc.VectorSubcoreMesh(
    core_axis_name="c", subcore_axis_name="s", num_cores=NC, num_subcores=NS)


def _sc_kernel_call(*args):
    return pl.kernel(
        _sc_aggr,
        out_type=jax.ShapeDtypeStruct((NC, NPAD, H), jnp.float32),
        mesh=_sc_mesh,
        interpret=_INTERPRET,
        scratch_types=[
            pltpu.VMEM((2, K), jnp.int32),
            pltpu.VMEM((2, K), jnp.int32),
            pltpu.VMEM((K,), jnp.float32),
            pltpu.VMEM((K,), jnp.float32),
            pltpu.VMEM((K,), jnp.int32),
            pltpu.VMEM((K, H), jnp.float32),
            pltpu.VMEM((K, H), jnp.float32),
            pltpu.VMEM((H,), jnp.float32),
            pltpu.VMEM((H,), jnp.float32),
            pltpu.VMEM_SHARED((NPAD, H), jnp.float32),
            pltpu.SemaphoreType.DMA,
            pltpu.SemaphoreType.DMA,
            pltpu.SemaphoreType.DMA,
            pltpu.SemaphoreType.DMA,
        ],
    )(*args)


def _sc_aggr(x_hbm, pk_hbm, av_hbm, vl_hbm, cl_hbm, out_hbm,
             pk0, pk1, av0, av1, dstv, rows0, rows1, vl_vm, cl_vm, aggr_sh,
             psem0, psem1, gsem0, gsem1):
    c = lax.axis_index("c")
    s = lax.axis_index("s")
    w = s * NC + c

    pltpu.sync_copy(vl_hbm, vl_vm)
    pltpu.sync_copy(cl_hbm, cl_vm)
    vlv = [vl_vm[pl.ds(16 * g, 16)] for g in range(8)]
    clv = [cl_vm[pl.ds(16 * g, 16)] for g in range(8)]

    z = jnp.zeros((16,), jnp.float32)

    def _zrow(e, carry):
        for g in range(8):
            rows0[e, pl.ds(16 * g, 16)] = z
        return carry

    lax.fori_loop(0, K, _zrow, 0)
    rbase = s * RPT
    for j in range(RPT // K):
        pltpu.sync_copy(rows0, aggr_sh.at[pl.ds(rbase + j * K, K)])
    plsc.subcore_barrier()

    def _compute(rows, avb):
        def _egroup(eg, ecarry):
            av = avb[pl.ds(16 * eg, 16)]
            for ei in range(16):
                ab = lax.broadcast(av[ei], (16,))
                e = 16 * eg + ei
                for g in range(8):
                    r = rows[e, pl.ds(16 * g, 16)]
                    rows[e, pl.ds(16 * g, 16)] = jnp.maximum(
                        r + ab * vlv[g] + clv[g], 0.0)
            return ecarry

        lax.fori_loop(0, K // 16, _egroup, 0)

    def _body(ch, pkP, pkQ, avP, avQ, psemP, psemQ,
              rowsP, rowsQ, gsemP, gsemQ):
        pltpu.make_async_copy(pk_hbm.at[w, ch], pkQ, psemQ).wait()
        pltpu.make_async_copy(av_hbm.at[w, ch], avQ, psemQ).wait()
        pltpu.async_copy(x_hbm.at[pkQ.at[0]], rowsQ, gsemQ)
        pltpu.make_async_copy(x_hbm.at[pkP.at[0]], rowsP, gsemP).wait()
        _compute(rowsP, avP)
        for i in range(K // 16):
            dstv[pl.ds(16 * i, 16)] = pkP[1, pl.ds(16 * i, 16)]
        pltpu.async_copy(pk_hbm.at[w, ch + 2], pkP, psemP)
        pltpu.async_copy(av_hbm.at[w, ch + 2], avP, psemP)
        pltpu.sync_copy(rowsP, aggr_sh.at[dstv], add=True)

    pltpu.async_copy(pk_hbm.at[w, 0], pk0, psem0).wait()
    pltpu.async_copy(av_hbm.at[w, 0], av0, psem0).wait()
    pltpu.async_copy(pk_hbm.at[w, 1], pk1, psem1)
    pltpu.async_copy(av_hbm.at[w, 1], av1, psem1)
    pltpu.async_copy(x_hbm.at[pk0.at[0]], rows0, gsem0)

    def _pair(t, carry):
        _body(2 * t, pk0, pk1, av0, av1, psem0, psem1,
              rows0, rows1, gsem0, gsem1)
        _body(2 * t + 1, pk1, pk0, av1, av0, psem1, psem0,
              rows1, rows0, gsem1, gsem0)
        return carry

    lax.fori_loop(0, (CHUNKS - 1) // 2, _pair, 0)

    pltpu.make_async_copy(pk_hbm.at[w, CHUNKS], pk1, psem1).wait()
    pltpu.make_async_copy(av_hbm.at[w, CHUNKS], av1, psem1).wait()
    pltpu.make_async_copy(x_hbm.at[pk0.at[0]], rows0, gsem0).wait()
    _compute(rows0, av0)
    for i in range(K // 16):
        dstv[pl.ds(16 * i, 16)] = pk0[1, pl.ds(16 * i, 16)]
    pltpu.sync_copy(rows0, aggr_sh.at[dstv], add=True)
    plsc.subcore_barrier()

    for j in range(RPT // K):
        sl = pl.ds(rbase + j * K, K)
        pltpu.sync_copy(aggr_sh.at[sl], rows0)
        pltpu.sync_copy(rows0, out_hbm.at[c, sl])


def _proj_body(nf_ref, Wi_ref, bi_ref, We_ref, be_ref, Wel_ref, bel_ref,
               x_ref, vc_ref):
    i = pl.program_id(0)
    x_ref[...] = (jnp.dot(nf_ref[...], Wi_ref[...],
                          preferred_element_type=jnp.float32,
                           precision=jax.lax.Precision.HIGHEST) + bi_ref[...])

    @pl.when(i == 0)
    def _():
        for l in range(3):
            Wl = Wel_ref[l]
            vr = jnp.dot(We_ref[...], Wl, preferred_element_type=jnp.float32,
                           precision=jax.lax.Precision.HIGHEST)
            cr = (jnp.dot(be_ref[...], Wl, preferred_element_type=jnp.float32,
                           precision=jax.lax.Precision.HIGHEST)
                  + bel_ref[pl.ds(l, 1), :])
            vc_ref[pl.ds(2 * l, 1), :] = vr
            vc_ref[pl.ds(2 * l + 1, 1), :] = cr
        vc_ref[pl.ds(6, 2), :] = jnp.zeros((2, H), jnp.float32)


def _proj(nf, Wi, bi2, We, be2, Wel, bel8):
    return pl.pallas_call(
        _proj_body,
        interpret=_INTERPRET,
        grid=(GRID,),
        in_specs=[
            pl.BlockSpec((BM, D_IN), lambda i: (i, 0)),
            pl.BlockSpec((D_IN, H), lambda i: (0, 0)),
            pl.BlockSpec((1, H), lambda i: (0, 0)),
            pl.BlockSpec((1, H), lambda i: (0, 0)),
            pl.BlockSpec((1, H), lambda i: (0, 0)),
            pl.BlockSpec((3, H, H), lambda i: (0, 0, 0)),
            pl.BlockSpec((8, H), lambda i: (0, 0)),
        ],
        out_specs=[
            pl.BlockSpec((BM, H), lambda i: (i, 0)),
            pl.BlockSpec((8, H), lambda i: (0, 0)),
        ],
        out_shape=[
            jax.ShapeDtypeStruct((NPAD, H), jnp.float32),
            jax.ShapeDtypeStruct((8, H), jnp.float32),
        ],
    )(nf, Wi, bi2, We, be2, Wel, bel8)


def _mlp_body(x_ref, agg_ref, W1_ref, b1_ref, W2_ref, b2_ref, o_ref):
    h = x_ref[...] + agg_ref[0] + agg_ref[1]
    t = jnp.maximum(jnp.dot(h, W1_ref[...],
                            preferred_element_type=jnp.float32,
                           precision=jax.lax.Precision.HIGHEST) + b1_ref[...],
                    0.0)
    y = jnp.dot(t, W2_ref[...], preferred_element_type=jnp.float32,
                           precision=jax.lax.Precision.HIGHEST) + b2_ref[...]
    o_ref[...] = jnp.maximum(y, 0.0)


def _mlp(x, agg, W1l, b1l2, W2l, b2l2):
    return pl.pallas_call(
        _mlp_body,
        interpret=_INTERPRET,
        grid=(GRID,),
        in_specs=[
            pl.BlockSpec((BM, H), lambda i: (i, 0)),
            pl.BlockSpec((2, BM, H), lambda i: (0, i, 0)),
            pl.BlockSpec((H, H), lambda i: (0, 0)),
            pl.BlockSpec((1, H), lambda i: (0, 0)),
            pl.BlockSpec((H, H), lambda i: (0, 0)),
            pl.BlockSpec((1, H), lambda i: (0, 0)),
        ],
        out_specs=pl.BlockSpec((BM, H), lambda i: (i, 0)),
        out_shape=jax.ShapeDtypeStruct((NPAD, H), jnp.float32),
    )(x, agg, W1l, b1l2, W2l, b2l2)


def _mlp3_body(x_ref, agg_ref, W1_ref, b1_ref, W2_ref, b2_ref, m_ref,
               o_ref, ps_ref, cnt_ref):
    i = pl.program_id(0)
    h = x_ref[...] + agg_ref[0] + agg_ref[1]
    t = jnp.maximum(jnp.dot(h, W1_ref[...],
                            preferred_element_type=jnp.float32,
                           precision=jax.lax.Precision.HIGHEST) + b1_ref[...],
                    0.0)
    y = jnp.dot(t, W2_ref[...], preferred_element_type=jnp.float32,
                           precision=jax.lax.Precision.HIGHEST) + b2_ref[...]
    xo = jnp.maximum(y, 0.0)
    o_ref[...] = xo
    oh = (jax.lax.broadcasted_iota(jnp.int32, (NF, BM), 0)
          == m_ref[0, 0, :][None, :]).astype(jnp.float32)
    ps = jax.lax.dot_general(oh, xo, (((1,), (0,)), ((), ())),
                             preferred_element_type=jnp.float32,
                             precision=jax.lax.Precision.HIGHEST)
    cb = jnp.broadcast_to(jnp.sum(oh, axis=1, keepdims=True), (NF, H))

    @pl.when(i == 0)
    def _():
        ps_ref[...] = ps
        cnt_ref[...] = cb

    @pl.when(i != 0)
    def _():
        ps_ref[...] = ps_ref[...] + ps
        cnt_ref[...] = cnt_ref[...] + cb


def _mlp3(x, agg, W1l, b1l2, W2l, b2l2, m3):
    return pl.pallas_call(
        _mlp3_body,
        interpret=_INTERPRET,
        grid=(GRID,),
        in_specs=[
            pl.BlockSpec((BM, H), lambda i: (i, 0)),
            pl.BlockSpec((2, BM, H), lambda i: (0, i, 0)),
            pl.BlockSpec((H, H), lambda i: (0, 0)),
            pl.BlockSpec((1, H), lambda i: (0, 0)),
            pl.BlockSpec((H, H), lambda i: (0, 0)),
            pl.BlockSpec((1, H), lambda i: (0, 0)),
            pl.BlockSpec((1, 1, BM), lambda i: (i, 0, 0)),
        ],
        out_specs=[
            pl.BlockSpec((BM, H), lambda i: (i, 0)),
            pl.BlockSpec((NF, H), lambda i: (0, 0)),
            pl.BlockSpec((NF, H), lambda i: (0, 0)),
        ],
        out_shape=[
            jax.ShapeDtypeStruct((NPAD, H), jnp.float32),
            jax.ShapeDtypeStruct((NF, H), jnp.float32),
            jax.ShapeDtypeStruct((NF, H), jnp.float32),
        ],
    )(x, agg, W1l, b1l2, W2l, b2l2, m3)


def _head_body(ps_ref, cnt_ref, Wh1_ref, bh1_ref, Wh2_ref, bh2_ref,
               Wh3_ref, bh3_ref, o_ref):
    pooled = ps_ref[...] / jnp.maximum(cnt_ref[...], 1.0)
    ctx = jnp.mean(pooled, axis=0, keepdims=True)
    A = Wh1_ref[pl.ds(0, H), :]
    B = Wh1_ref[pl.ds(H, H), :]
    h1 = jnp.maximum(
        jnp.dot(pooled, A, preferred_element_type=jnp.float32,
                           precision=jax.lax.Precision.HIGHEST)
        + jnp.dot(ctx, B, preferred_element_type=jnp.float32,
                           precision=jax.lax.Precision.HIGHEST)
        + bh1_ref[...], 0.0)
    h2 = jnp.maximum(
        jnp.dot(h1, Wh2_ref[...], preferred_element_type=jnp.float32,
                           precision=jax.lax.Precision.HIGHEST)
        + bh2_ref[...], 0.0)
    o_ref[...] = (jnp.dot(h2, Wh3_ref[...], preferred_element_type=jnp.float32,
                           precision=jax.lax.Precision.HIGHEST)
                  + bh3_ref[...])


def _head(ps, cnt, Wh1, bh12, Wh2, bh22, Wh3, bh32):
    return pl.pallas_call(
        _head_body,
        interpret=_INTERPRET,
        out_shape=jax.ShapeDtypeStruct((NF, 1), jnp.float32),
    )(ps, cnt, Wh1, bh12, Wh2, bh22, Wh3, bh32)


def kernel(node_features, edge_index, edge_attr, membership,
           Wi, bi, We, be, Wel, bel, W1, b1, W2, b2,
           Wh1, bh1, Wh2, bh2, Wh3, bh3):
    f32 = jnp.float32
    nf = node_features.astype(f32)
    bel8 = jnp.concatenate([bel.astype(f32), jnp.zeros((5, H), f32)], axis=0)
    x, vc = _proj(nf, Wi.astype(f32), bi.reshape(1, H).astype(f32),
                  We.astype(f32), be.reshape(1, H).astype(f32),
                  Wel.astype(f32), bel8)

    src3 = edge_index[0].reshape(NTILES, CHUNKS, K)
    dst3 = edge_index[1].reshape(NTILES, CHUNKS, K)
    a3 = edge_attr.astype(f32)[:, 0].reshape(NTILES, CHUNKS, K)
    av3 = jnp.concatenate(
        [a3, jnp.zeros((NTILES, 1, K), f32)], axis=1)
    pk = jnp.stack([src3, dst3], axis=2)
    pk = jnp.concatenate(
        [pk, jnp.zeros((NTILES, 1, 2, K), jnp.int32)], axis=1)
    m3 = membership.reshape(GRID, 1, BM)

    for l in range(3):
        agg = _sc_kernel_call(x, pk, av3, vc[2 * l], vc[2 * l + 1])
        W1l = W1[l].astype(f32)
        b1l2 = b1[l].reshape(1, H).astype(f32)
        W2l = W2[l].astype(f32)
        b2l2 = b2[l].reshape(1, H).astype(f32)
        if l < 2:
            x = _mlp(x, agg, W1l, b1l2, W2l, b2l2)
        else:
            x, ps, cnt = _mlp3(x, agg, W1l, b1l2, W2l, b2l2, m3)

    return _head(ps, cnt, Wh1.astype(f32), bh1.reshape(1, H).astype(f32),
                 Wh2.astype(f32), bh2.reshape(1, H).astype(f32),
                 Wh3.astype(f32), bh3.reshape(1, 1).astype(f32))


# speedup vs baseline: 8.2970x; 1.0368x over previous
import functools

_INTERPRET = False

import jax
import jax.numpy as jnp
from jax import lax
from jax.experimental import pallas as pl
from jax.experimental.pallas import tpu as pltpu
from jax.experimental.pallas import tpu_sc as plsc

N = 10000
E = 320000
D_IN = 128
H = 128
NF = 512
NPAD = 10240
NC, NS = 2, 16
NTILES = NC * NS
EPT = E // NTILES
K = 80
CHUNKS = EPT // K
RPT = NPAD // NS
BM = 2000
GRID = N // BM


_sc_mesh = plsc.VectorSubcoreMesh(
    core_axis_name="c", subcore_axis_name="s", num_cores=NC, num_subcores=NS)


def _sc_kernel_call(*args):
    return pl.kernel(
        _sc_aggr,
        out_type=jax.ShapeDtypeStruct((NC, NPAD, H), jnp.float32),
        mesh=_sc_mesh,
        interpret=_INTERPRET,
        scratch_types=[
            pltpu.VMEM((2, K), jnp.int32),
            pltpu.VMEM((2, K), jnp.int32),
            pltpu.VMEM((K,), jnp.float32),
            pltpu.VMEM((K,), jnp.float32),
            pltpu.VMEM((K,), jnp.int32),
            pltpu.VMEM((K, H), jnp.float32),
            pltpu.VMEM((K, H), jnp.float32),
            pltpu.VMEM((H,), jnp.float32),
            pltpu.VMEM_SHARED((NPAD, H), jnp.float32),
            pltpu.SemaphoreType.DMA,
            pltpu.SemaphoreType.DMA,
            pltpu.SemaphoreType.DMA,
            pltpu.SemaphoreType.DMA,
        ],
    )(*args)


def _sc_aggr(x_hbm, pk_hbm, av_hbm, vl_hbm, out_hbm,
             pk0, pk1, av0, av1, dstv, rows0, rows1, vl_vm, aggr_sh,
             psem0, psem1, gsem0, gsem1):
    c = lax.axis_index("c")
    s = lax.axis_index("s")
    w = s * NC + c

    pltpu.sync_copy(vl_hbm, vl_vm)
    vlv = [vl_vm[pl.ds(16 * g, 16)] for g in range(8)]

    z = jnp.zeros((16,), jnp.float32)

    def _zrow(e, carry):
        for g in range(8):
            rows0[e, pl.ds(16 * g, 16)] = z
        return carry

    lax.fori_loop(0, K, _zrow, 0)
    rbase = s * RPT
    for j in range(RPT // K):
        pltpu.sync_copy(rows0, aggr_sh.at[pl.ds(rbase + j * K, K)])
    plsc.subcore_barrier()

    def _compute(rows, avb):
        def _egroup(eg, ecarry):
            av = avb[pl.ds(16 * eg, 16)]
            for ei in range(16):
                ab = lax.broadcast(av[ei], (16,))
                e = 16 * eg + ei
                for g in range(8):
                    r = rows[e, pl.ds(16 * g, 16)]
                    rows[e, pl.ds(16 * g, 16)] = jnp.maximum(
                        r + ab * vlv[g], 0.0)
            return ecarry

        lax.fori_loop(0, K // 16, _egroup, 0)

    def _body(ch, pkP, pkQ, avP, avQ, psemP, psemQ,
              rowsP, rowsQ, gsemP, gsemQ):
        pltpu.make_async_copy(pk_hbm.at[w, ch], pkQ, psemQ).wait()
        pltpu.make_async_copy(av_hbm.at[w, ch], avQ, psemQ).wait()
        pltpu.async_copy(x_hbm.at[pkQ.at[0]], rowsQ, gsemQ)
        pltpu.make_async_copy(x_hbm.at[pkP.at[0]], rowsP, gsemP).wait()
        _compute(rowsP, avP)
        for i in range(K // 16):
            dstv[pl.ds(16 * i, 16)] = pkP[1, pl.ds(16 * i, 16)]
        pltpu.async_copy(pk_hbm.at[w, ch + 2], pkP, psemP)
        pltpu.async_copy(av_hbm.at[w, ch + 2], avP, psemP)
        pltpu.sync_copy(rowsP, aggr_sh.at[dstv], add=True)

    pltpu.async_copy(pk_hbm.at[w, 0], pk0, psem0).wait()
    pltpu.async_copy(av_hbm.at[w, 0], av0, psem0).wait()
    pltpu.async_copy(pk_hbm.at[w, 1], pk1, psem1)
    pltpu.async_copy(av_hbm.at[w, 1], av1, psem1)
    pltpu.async_copy(x_hbm.at[pk0.at[0]], rows0, gsem0)

    def _pair(t, carry):
        _body(2 * t, pk0, pk1, av0, av1, psem0, psem1,
              rows0, rows1, gsem0, gsem1)
        _body(2 * t + 1, pk1, pk0, av1, av0, psem1, psem0,
              rows1, rows0, gsem1, gsem0)
        return carry

    lax.fori_loop(0, (CHUNKS - 1) // 2, _pair, 0)

    pltpu.make_async_copy(pk_hbm.at[w, CHUNKS], pk1, psem1).wait()
    pltpu.make_async_copy(av_hbm.at[w, CHUNKS], av1, psem1).wait()
    pltpu.make_async_copy(x_hbm.at[pk0.at[0]], rows0, gsem0).wait()
    _compute(rows0, av0)
    for i in range(K // 16):
        dstv[pl.ds(16 * i, 16)] = pk0[1, pl.ds(16 * i, 16)]
    pltpu.sync_copy(rows0, aggr_sh.at[dstv], add=True)
    plsc.subcore_barrier()

    for j in range(RPT // K):
        sl = pl.ds(rbase + j * K, K)
        pltpu.sync_copy(aggr_sh.at[sl], rows0)
        pltpu.sync_copy(rows0, out_hbm.at[c, sl])


def _proj_body(nf_ref, Wi_ref, bi_ref, We_ref, be_ref, Wel_ref, bel_ref,
               x_ref, xc_ref, vc_ref):
    i = pl.program_id(0)
    xb = (jnp.dot(nf_ref[...], Wi_ref[...],
                  preferred_element_type=jnp.float32,
                  precision=jax.lax.Precision.HIGHEST) + bi_ref[...])
    x_ref[...] = xb
    cl0 = (jnp.dot(be_ref[...], Wel_ref[0],
                   preferred_element_type=jnp.float32,
                   precision=jax.lax.Precision.HIGHEST)
           + bel_ref[pl.ds(0, 1), :])
    xc_ref[...] = xb + cl0

    @pl.when(i == 0)
    def _():
        for l in range(3):
            Wl = Wel_ref[l]
            vr = jnp.dot(We_ref[...], Wl, preferred_element_type=jnp.float32,
                           precision=jax.lax.Precision.HIGHEST)
            cr = (jnp.dot(be_ref[...], Wl, preferred_element_type=jnp.float32,
                           precision=jax.lax.Precision.HIGHEST)
                  + bel_ref[pl.ds(l, 1), :])
            vc_ref[pl.ds(2 * l, 1), :] = vr
            vc_ref[pl.ds(2 * l + 1, 1), :] = cr
        vc_ref[pl.ds(6, 2), :] = jnp.zeros((2, H), jnp.float32)


def _proj(nf, Wi, bi2, We, be2, Wel, bel8):
    return pl.pallas_call(
        _proj_body,
        interpret=_INTERPRET,
        grid=(GRID,),
        in_specs=[
            pl.BlockSpec((BM, D_IN), lambda i: (i, 0)),
            pl.BlockSpec((D_IN, H), lambda i: (0, 0)),
            pl.BlockSpec((1, H), lambda i: (0, 0)),
            pl.BlockSpec((1, H), lambda i: (0, 0)),
            pl.BlockSpec((1, H), lambda i: (0, 0)),
            pl.BlockSpec((3, H, H), lambda i: (0, 0, 0)),
            pl.BlockSpec((8, H), lambda i: (0, 0)),
        ],
        out_specs=[
            pl.BlockSpec((BM, H), lambda i: (i, 0)),
            pl.BlockSpec((BM, H), lambda i: (i, 0)),
            pl.BlockSpec((8, H), lambda i: (0, 0)),
        ],
        out_shape=[
            jax.ShapeDtypeStruct((NPAD, H), jnp.float32),
            jax.ShapeDtypeStruct((NPAD, H), jnp.float32),
            jax.ShapeDtypeStruct((8, H), jnp.float32),
        ],
    )(nf, Wi, bi2, We, be2, Wel, bel8)


def _mlp_body(x_ref, agg_ref, W1_ref, b1_ref, W2_ref, b2_ref, cn_ref,
              o_ref, oc_ref):
    h = x_ref[...] + agg_ref[0] + agg_ref[1]
    t = jnp.maximum(jnp.dot(h, W1_ref[...],
                            preferred_element_type=jnp.float32,
                            precision=jax.lax.Precision.HIGHEST) + b1_ref[...],
                    0.0)
    y = jnp.dot(t, W2_ref[...], preferred_element_type=jnp.float32,
                precision=jax.lax.Precision.HIGHEST) + b2_ref[...]
    xo = jnp.maximum(y, 0.0)
    o_ref[...] = xo
    oc_ref[...] = xo + cn_ref[...]


def _mlp(x, agg, W1l, b1l2, W2l, b2l2, cn2):
    return pl.pallas_call(
        _mlp_body,
        interpret=_INTERPRET,
        grid=(GRID,),
        in_specs=[
            pl.BlockSpec((BM, H), lambda i: (i, 0)),
            pl.BlockSpec((2, BM, H), lambda i: (0, i, 0)),
            pl.BlockSpec((H, H), lambda i: (0, 0)),
            pl.BlockSpec((1, H), lambda i: (0, 0)),
            pl.BlockSpec((H, H), lambda i: (0, 0)),
            pl.BlockSpec((1, H), lambda i: (0, 0)),
            pl.BlockSpec((1, H), lambda i: (0, 0)),
        ],
        out_specs=[
            pl.BlockSpec((BM, H), lambda i: (i, 0)),
            pl.BlockSpec((BM, H), lambda i: (i, 0)),
        ],
        out_shape=[
            jax.ShapeDtypeStruct((NPAD, H), jnp.float32),
            jax.ShapeDtypeStruct((NPAD, H), jnp.float32),
        ],
    )(x, agg, W1l, b1l2, W2l, b2l2, cn2)


def _mlp3_body(x_ref, agg_ref, W1_ref, b1_ref, W2_ref, b2_ref, m_ref,
               o_ref, ps_ref, cnt_ref):
    i = pl.program_id(0)
    h = x_ref[...] + agg_ref[0] + agg_ref[1]
    t = jnp.maximum(jnp.dot(h, W1_ref[...],
                            preferred_element_type=jnp.float32,
                           precision=jax.lax.Precision.HIGHEST) + b1_ref[...],
                    0.0)
    y = jnp.dot(t, W2_ref[...], preferred_element_type=jnp.float32,
                           precision=jax.lax.Precision.HIGHEST) + b2_ref[...]
    xo = jnp.maximum(y, 0.0)
    o_ref[...] = xo
    oh = (jax.lax.broadcasted_iota(jnp.int32, (NF, BM), 0)
          == m_ref[0, 0, :][None, :]).astype(jnp.float32)
    ps = jax.lax.dot_general(oh, xo, (((1,), (0,)), ((), ())),
                             preferred_element_type=jnp.float32,
                             precision=jax.lax.Precision.HIGHEST)
    cb = jnp.broadcast_to(jnp.sum(oh, axis=1, keepdims=True), (NF, H))

    @pl.when(i == 0)
    def _():
        ps_ref[...] = ps
        cnt_ref[...] = cb

    @pl.when(i != 0)
    def _():
        ps_ref[...] = ps_ref[...] + ps
        cnt_ref[...] = cnt_ref[...] + cb


def _mlp3(x, agg, W1l, b1l2, W2l, b2l2, m3):
    return pl.pallas_call(
        _mlp3_body,
        interpret=_INTERPRET,
        grid=(GRID,),
        in_specs=[
            pl.BlockSpec((BM, H), lambda i: (i, 0)),
            pl.BlockSpec((2, BM, H), lambda i: (0, i, 0)),
            pl.BlockSpec((H, H), lambda i: (0, 0)),
            pl.BlockSpec((1, H), lambda i: (0, 0)),
            pl.BlockSpec((H, H), lambda i: (0, 0)),
            pl.BlockSpec((1, H), lambda i: (0, 0)),
            pl.BlockSpec((1, 1, BM), lambda i: (i, 0, 0)),
        ],
        out_specs=[
            pl.BlockSpec((BM, H), lambda i: (i, 0)),
            pl.BlockSpec((NF, H), lambda i: (0, 0)),
            pl.BlockSpec((NF, H), lambda i: (0, 0)),
        ],
        out_shape=[
            jax.ShapeDtypeStruct((NPAD, H), jnp.float32),
            jax.ShapeDtypeStruct((NF, H), jnp.float32),
            jax.ShapeDtypeStruct((NF, H), jnp.float32),
        ],
    )(x, agg, W1l, b1l2, W2l, b2l2, m3)


def _head_body(ps_ref, cnt_ref, Wh1_ref, bh1_ref, Wh2_ref, bh2_ref,
               Wh3_ref, bh3_ref, o_ref):
    pooled = ps_ref[...] / jnp.maximum(cnt_ref[...], 1.0)
    ctx = jnp.mean(pooled, axis=0, keepdims=True)
    A = Wh1_ref[pl.ds(0, H), :]
    B = Wh1_ref[pl.ds(H, H), :]
    h1 = jnp.maximum(
        jnp.dot(pooled, A, preferred_element_type=jnp.float32,
                           precision=jax.lax.Precision.HIGHEST)
        + jnp.dot(ctx, B, preferred_element_type=jnp.float32,
                           precision=jax.lax.Precision.HIGHEST)
        + bh1_ref[...], 0.0)
    h2 = jnp.maximum(
        jnp.dot(h1, Wh2_ref[...], preferred_element_type=jnp.float32,
                           precision=jax.lax.Precision.HIGHEST)
        + bh2_ref[...], 0.0)
    o_ref[...] = (jnp.dot(h2, Wh3_ref[...], preferred_element_type=jnp.float32,
                           precision=jax.lax.Precision.HIGHEST)
                  + bh3_ref[...])


def _head(ps, cnt, Wh1, bh12, Wh2, bh22, Wh3, bh32):
    return pl.pallas_call(
        _head_body,
        interpret=_INTERPRET,
        out_shape=jax.ShapeDtypeStruct((NF, 1), jnp.float32),
    )(ps, cnt, Wh1, bh12, Wh2, bh22, Wh3, bh32)


def kernel(node_features, edge_index, edge_attr, membership,
           Wi, bi, We, be, Wel, bel, W1, b1, W2, b2,
           Wh1, bh1, Wh2, bh2, Wh3, bh3):
    f32 = jnp.float32
    nf = node_features.astype(f32)
    bel8 = jnp.concatenate([bel.astype(f32), jnp.zeros((5, H), f32)], axis=0)
    x, xc, vc = _proj(nf, Wi.astype(f32), bi.reshape(1, H).astype(f32),
                      We.astype(f32), be.reshape(1, H).astype(f32),
                      Wel.astype(f32), bel8)

    src3 = edge_index[0].reshape(NTILES, CHUNKS, K)
    dst3 = edge_index[1].reshape(NTILES, CHUNKS, K)
    a3 = edge_attr.astype(f32)[:, 0].reshape(NTILES, CHUNKS, K)
    av3 = jnp.concatenate(
        [a3, jnp.zeros((NTILES, 1, K), f32)], axis=1)
    pk = jnp.stack([src3, dst3], axis=2)
    pk = jnp.concatenate(
        [pk, jnp.zeros((NTILES, 1, 2, K), jnp.int32)], axis=1)
    m3 = membership.reshape(GRID, 1, BM)

    for l in range(3):
        agg = _sc_kernel_call(xc, pk, av3, vc[2 * l])
        W1l = W1[l].astype(f32)
        b1l2 = b1[l].reshape(1, H).astype(f32)
        W2l = W2[l].astype(f32)
        b2l2 = b2[l].reshape(1, H).astype(f32)
        if l < 2:
            cn2 = vc[2 * l + 3].reshape(1, H)
            x, xc = _mlp(x, agg, W1l, b1l2, W2l, b2l2, cn2)
        else:
            x, ps, cnt = _mlp3(x, agg, W1l, b1l2, W2l, b2l2, m3)

    return _head(ps, cnt, Wh1.astype(f32), bh1.reshape(1, H).astype(f32),
                 Wh2.astype(f32), bh2.reshape(1, H).astype(f32),
                 Wh3.astype(f32), bh3.reshape(1, 1).astype(f32))
